# Initial kernel scaffold; baseline (speedup 1.0000x reference)
import jax
import jax.numpy as jnp
from jax.experimental import pallas as pl


def kernel(input, index, src, output):
    raise NotImplementedError("write your pallas kernel here")


# speedup vs baseline: 1.4875x; 1.4875x over previous
import jax
import jax.numpy as jnp
from jax import lax
from jax.experimental import pallas as pl
from jax.experimental.pallas import tpu as pltpu
from jax.experimental.pallas import tpu_sc as plsc

_M, _D, _B = 1000000, 64, 16384
_NC, _NS = 2, 16
_NW = _NC * _NS
_SUB_ROWS = 800
_NBLK = _M // _SUB_ROWS
_SUB_W = _SUB_ROWS * _D
_CB = 64
_NCHUNK = _B // _CB
_FLUSH = 256
_ABUF = _FLUSH + 64
_PC = 1024
_CAPH = _B * _D + _FLUSH + _PC


def _sc_body(idx_hbm, src_hbm, inp_hbm, res_hbm, pi_hbm, pv_hbm,
             idx_c, src_c, a_i, a_v, blk, p_i, p_v):
    wid = lax.axis_index("s") * _NC + lax.axis_index("c")
    lanes = lax.iota(jnp.int32, 16)
    wid_v = jnp.full((16,), wid, jnp.int32)

    def chunk_body(c, carry):
        pltpu.sync_copy(idx_hbm.at[pl.ds(c * _CB, _CB)], idx_c)
        pltpu.sync_copy(src_hbm.at[pl.ds(c * _CB, _CB)], src_c)

        def row_body(i, carry2):
            n, off = carry2
            for j in range(4):
                iv = idx_c[i, pl.ds(j * 16, 16)]
                sv = src_c[i, pl.ds(j * 16, 16)]
                bid = lax.div(iv, jnp.full((16,), _SUB_ROWS, jnp.int32))
                m = lax.rem(bid, jnp.full((16,), _NW, jnp.int32)) == wid_v
                flat = (lax.div(bid, jnp.full((16,), _NW, jnp.int32)) * _SUB_W
                        + (iv - bid * _SUB_ROWS) * _D + (j * 16 + lanes))
                ones = jnp.full((16,), 1, jnp.int32)
                zeros = jnp.full((16,), 0, jnp.int32)
                x = jnp.where(m, ones, zeros)
                for k in (1, 2, 4, 8):
                    g = jnp.maximum(lanes - k, zeros)
                    xs = x.at[g].get(mode="promise_in_bounds")
                    x = x + jnp.where(lanes >= k, xs, zeros)
                pos = jnp.full((16,), n, jnp.int32) + x - 1
                plsc.store_scatter(a_i, [pos], flat, mask=m)
                plsc.store_scatter(a_v, [pos], sv, mask=m)
                n = n + plsc.all_reduce_population_count(m)[0]

            def do_flush(c3):
                n3, off3 = c3
                dst = pl.multiple_of(wid * _CAPH + off3, 256)
                pltpu.sync_copy(a_i.at[pl.ds(0, _FLUSH)],
                                pi_hbm.at[pl.ds(dst, _FLUSH)])
                pltpu.sync_copy(a_v.at[pl.ds(0, _FLUSH)],
                                pv_hbm.at[pl.ds(dst, _FLUSH)])
                for t in range(4):
                    li = a_i[pl.ds(_FLUSH + t * 16, 16)]
                    lv = a_v[pl.ds(_FLUSH + t * 16, 16)]
                    a_i[pl.ds(t * 16, 16)] = li
                    a_v[pl.ds(t * 16, 16)] = lv
                return (n3 - _FLUSH, off3 + _FLUSH)

            return lax.cond(n >= _FLUSH, do_flush, lambda c3: c3, (n, off))

        return lax.fori_loop(0, _CB, row_body, carry)

    n, off = lax.fori_loop(0, _NCHUNK, chunk_body,
                           (jnp.int32(0), jnp.int32(0)))

    for t in range(_ABUF // 16):
        posv = t * 16 + lanes
        mj = posv >= jnp.full((16,), n, jnp.int32)
        a_i[pl.ds(t * 16, 16)] = jnp.where(mj, 0, a_i[pl.ds(t * 16, 16)])
        a_v[pl.ds(t * 16, 16)] = jnp.where(mj, 0.0, a_v[pl.ds(t * 16, 16)])
    dst = pl.multiple_of(wid * _CAPH + off, 256)
    pltpu.sync_copy(a_i.at[pl.ds(0, _FLUSH)], pi_hbm.at[pl.ds(dst, _FLUSH)])
    pltpu.sync_copy(a_v.at[pl.ds(0, _FLUSH)], pv_hbm.at[pl.ds(dst, _FLUSH)])
    total = off + _FLUSH

    nchunk2 = (total + _PC - 1) // _PC
    nblk_w = jnp.int32(_NBLK // _NW) + (wid < (_NBLK % _NW)).astype(jnp.int32)

    def sub_body(s, _):
        row0 = pl.multiple_of((s * _NW + wid) * _SUB_ROWS, 8)
        flat_lo = s * _SUB_W
        pltpu.sync_copy(inp_hbm.at[pl.ds(row0, _SUB_ROWS)], blk)

        def pchunk_body(ch, _2):
            src0 = pl.multiple_of(wid * _CAPH + ch * _PC, 256)
            pltpu.sync_copy(pi_hbm.at[pl.ds(src0, _PC)], p_i)
            pltpu.sync_copy(pv_hbm.at[pl.ds(src0, _PC)], p_v)

            def pvec_body(v, _3):
                fv = p_i[pl.ds(v * 16, 16)]
                vv = p_v[pl.ds(v * 16, 16)]
                gpos = jnp.full((16,), ch * _PC + v * 16, jnp.int32) + lanes
                flat_lo_v = jnp.full((16,), flat_lo, jnp.int32)
                m = ((fv >= flat_lo_v) & (fv < flat_lo_v + _SUB_W)
                     & (gpos < jnp.full((16,), total, jnp.int32)))
                q = jnp.where(m, fv - flat_lo_v, 0)
                r = lax.div(q, jnp.full((16,), _D, jnp.int32))
                cc = lax.rem(q, jnp.full((16,), _D, jnp.int32))
                for lane in range(16):
                    plsc.addupdate_scatter(blk, [r, cc], vv,
                                           mask=m & (lanes == lane))
                return 0

            return lax.fori_loop(0, _PC // 16, pvec_body, 0)

        lax.fori_loop(0, nchunk2, pchunk_body, 0)
        pltpu.sync_copy(blk, res_hbm.at[pl.ds(row0, _SUB_ROWS)])
        return 0

    lax.fori_loop(0, nblk_w, sub_body, 0)


@jax.jit
def _scatter_add(inp, index, src):
    mesh = plsc.VectorSubcoreMesh(core_axis_name="c", subcore_axis_name="s")
    fn = pl.kernel(
        _sc_body,
        out_type=(
            jax.ShapeDtypeStruct((_M, _D), jnp.float32),
            jax.ShapeDtypeStruct((_NW * _CAPH,), jnp.int32),
            jax.ShapeDtypeStruct((_NW * _CAPH,), jnp.float32),
        ),
        mesh=mesh,
        compiler_params=pltpu.CompilerParams(needs_layout_passes=False),
        scratch_types=[
            pltpu.VMEM((_CB, _D), jnp.int32),
            pltpu.VMEM((_CB, _D), jnp.float32),
            pltpu.VMEM((_ABUF,), jnp.int32),
            pltpu.VMEM((_ABUF,), jnp.float32),
            pltpu.VMEM((_SUB_ROWS, _D), jnp.float32),
            pltpu.VMEM((_PC,), jnp.int32),
            pltpu.VMEM((_PC,), jnp.float32),
        ],
    )
    res, _, _ = fn(index, src, inp)
    return res


def kernel(input, index, src, output):
    result = _scatter_add(input, index, src)
    return (input, index, src, result)
